# baseline (speedup 1.0000x reference)
import math

import jax
import jax.numpy as jnp
import numpy as np
from jax.experimental import pallas as pl
from jax.sharding import Mesh, PartitionSpec as P

DIM = 1024
HEADS = 16
BLOCK = 128
DH = DIM // HEADS
SEQ = 2048
NMODS = 3

TB = 256
FTB = 512


def _block_attn_kernel(x_ref, wqkv_ref, wo_ref, bqkv_ref, bo_ref, o_ref):
    f32 = jnp.float32
    bf16 = jnp.bfloat16
    x = x_ref[0]
    qkv = jnp.dot(x, wqkv_ref[0], preferred_element_type=f32) + bqkv_ref[0]
    qb = qkv[:, :DIM].astype(bf16)
    kb = qkv[:, DIM:2 * DIM].astype(bf16)
    vb = qkv[:, 2 * DIM:].astype(bf16)
    nsb = TB // BLOCK
    scores = []
    for s in range(nsb):
        qs = qb[s * BLOCK:(s + 1) * BLOCK]
        ks = kb[s * BLOCK:(s + 1) * BLOCK]
        for h in range(HEADS):
            qh = qs[:, h * DH:(h + 1) * DH]
            kh = ks[:, h * DH:(h + 1) * DH]
            scores.append(jax.lax.dot_general(
                qh, kh, (((1,), (1,)), ((), ())),
                preferred_element_type=f32))
    sc = jnp.concatenate(scores, axis=0)
    m = jnp.max(sc, axis=-1, keepdims=True)
    e = jnp.exp(sc - m)
    p = e / jnp.sum(e, axis=-1, keepdims=True)
    pb = p.astype(bf16)
    row_blocks = []
    for s in range(nsb):
        vs = vb[s * BLOCK:(s + 1) * BLOCK]
        heads = []
        for h in range(HEADS):
            ph = pb[(s * HEADS + h) * BLOCK:(s * HEADS + h + 1) * BLOCK]
            vh = vs[:, h * DH:(h + 1) * DH]
            heads.append(jnp.dot(ph, vh, preferred_element_type=f32))
        row_blocks.append(jnp.concatenate(heads, axis=-1))
    att = jnp.concatenate(row_blocks, axis=0)
    o = jnp.dot(att.astype(bf16), wo_ref[0],
                preferred_element_type=f32) + bo_ref[0]
    o_ref[0] = o.astype(o_ref.dtype)


def _block_attn(x, wqkv, wo, bqkv, bo):
    m, s, _ = x.shape
    ntb = s // TB
    return pl.pallas_call(
        _block_attn_kernel,
        grid=(m, ntb),
        in_specs=[
            pl.BlockSpec((1, TB, DIM), lambda i, j: (i, j, 0)),
            pl.BlockSpec((1, DIM, 3 * DIM), lambda i, j: (i, 0, 0)),
            pl.BlockSpec((1, DIM, DIM), lambda i, j: (i, 0, 0)),
            pl.BlockSpec((1, 1, 3 * DIM), lambda i, j: (i, 0, 0)),
            pl.BlockSpec((1, 1, DIM), lambda i, j: (i, 0, 0)),
        ],
        out_specs=pl.BlockSpec((1, TB, DIM), lambda i, j: (i, j, 0)),
        out_shape=jax.ShapeDtypeStruct((m, s, DIM), jnp.bfloat16),
    )(x, wqkv, wo, bqkv, bo)


def _fusion_kernel(a_ref, c_ref, w_ref, b_ref, o_ref):
    f32 = jnp.float32
    acc = jnp.dot(a_ref[0], w_ref[0], preferred_element_type=f32)
    for i in range(1, NMODS):
        acc += jnp.dot(a_ref[i], w_ref[i], preferred_element_type=f32)
    for i in range(NMODS):
        acc += jnp.dot(c_ref[i], w_ref[NMODS + i], preferred_element_type=f32)
    o_ref[...] = acc + b_ref[...]


def _fusion(a, c, wf, bf):
    s = a.shape[1]
    ftb = min(FTB, s)
    nt = s // ftb
    return pl.pallas_call(
        _fusion_kernel,
        grid=(nt,),
        in_specs=[
            pl.BlockSpec((NMODS, ftb, DIM), lambda i: (0, i, 0)),
            pl.BlockSpec((NMODS, ftb, DIM), lambda i: (0, i, 0)),
            pl.BlockSpec((2 * NMODS, DIM, DIM), lambda i: (0, 0, 0)),
            pl.BlockSpec((1, DIM), lambda i: (0, 0)),
        ],
        out_specs=pl.BlockSpec((ftb, DIM), lambda i: (i, 0)),
        out_shape=jax.ShapeDtypeStruct((s, DIM), jnp.float32),
    )(a, c, wf, bf)


def _attn_operands(plist):
    scale = 1.0 / math.sqrt(DH)
    wqkv = jnp.stack([
        jnp.concatenate([p["Wq"] * scale, p["Wk"], p["Wv"]], axis=1)
        for p in plist]).astype(jnp.bfloat16)
    wo = jnp.stack([p["Wo"] for p in plist]).astype(jnp.bfloat16)
    bqkv = jnp.stack([
        jnp.concatenate([p["bq"] * scale, p["bk"], p["bv"]])
        for p in plist]).astype(jnp.float32).reshape(len(plist), 1, 3 * DIM)
    bo = jnp.stack([p["bo"] for p in plist]).astype(
        jnp.float32).reshape(len(plist), 1, DIM)
    return wqkv, wo, bqkv, bo


def _pipeline(x, wqkv_m, wo_m, bqkv_m, bo_m,
              wqkv_c, wo_c, bqkv_c, bo_c, wf, bfus):
    s_local = x.shape[1]
    attended = _block_attn(x, wqkv_m, wo_m, bqkv_m, bo_m)
    cross = _block_attn(attended.reshape(1, NMODS * s_local, DIM),
                        wqkv_c, wo_c, bqkv_c, bo_c)
    cross = cross.reshape(NMODS, s_local, DIM)
    return _fusion(attended, cross, wf, bfus)


def kernel(text, visual, audio, params):
    bf16 = jnp.bfloat16
    x = jnp.stack([text[0], visual[0], audio[0]]).astype(bf16)
    mod_ops = _attn_operands([params[m + "_attn"]
                              for m in ("text", "visual", "audio")])
    cross_ops = _attn_operands([params["cross_attn"]])
    fw = params["fusion_weights"].astype(jnp.float32)
    scales = jnp.concatenate([fw, fw]).reshape(2 * NMODS, 1, 1)
    wf = (params["fusion_W"].reshape(2 * NMODS, DIM, DIM) * scales).astype(bf16)
    bfus = params["fusion_b"].astype(jnp.float32).reshape(1, DIM)

    devs = jax.devices()
    nd = 2 if len(devs) >= 2 and SEQ % (2 * TB) == 0 else 1
    if nd == 1:
        out = _pipeline(x, *mod_ops, *cross_ops, wf, bfus)
        return out.reshape(1, SEQ, DIM)

    mesh = Mesh(np.array(devs[:nd]), ("x",))
    rep = (P(),) * 10
    out = jax.shard_map(
        _pipeline, mesh=mesh,
        in_specs=(P(None, "x", None),) + rep,
        out_specs=P("x", None),
        check_vma=False,
    )(x, *mod_ops, *cross_ops, wf, bfus)
    return out.reshape(1, SEQ, DIM)


# speedup vs baseline: 3.3972x; 3.3972x over previous
import math

import jax
import jax.numpy as jnp
from jax.experimental import pallas as pl

DIM = 1024
HEADS = 16
BLOCK = 128
DH = DIM // HEADS
SEQ = 2048
NMODS = 3

TB = 256
FTB = 512


def _block_attn_kernel(x_ref, wqkv_ref, wo_ref, bqkv_ref, bo_ref, o_ref):
    f32 = jnp.float32
    bf16 = jnp.bfloat16
    x = x_ref[0]
    qkv = jnp.dot(x, wqkv_ref[0], preferred_element_type=f32) + bqkv_ref[0]
    qb = qkv[:, :DIM].astype(bf16)
    kb = qkv[:, DIM:2 * DIM].astype(bf16)
    vb = qkv[:, 2 * DIM:].astype(bf16)
    nsb = TB // BLOCK
    scores = []
    for s in range(nsb):
        qs = qb[s * BLOCK:(s + 1) * BLOCK]
        ks = kb[s * BLOCK:(s + 1) * BLOCK]
        for h in range(HEADS):
            qh = qs[:, h * DH:(h + 1) * DH]
            kh = ks[:, h * DH:(h + 1) * DH]
            scores.append(jax.lax.dot_general(
                kh, qh, (((1,), (1,)), ((), ())),
                preferred_element_type=f32))
    sc = jnp.concatenate(scores, axis=1)
    m = jnp.max(sc, axis=0, keepdims=True)
    e = jnp.exp(sc - m)
    p = e * (1.0 / jnp.sum(e, axis=0, keepdims=True))
    pb = p.astype(bf16)
    row_blocks = []
    for s in range(nsb):
        vs = vb[s * BLOCK:(s + 1) * BLOCK]
        heads = []
        for h in range(HEADS):
            ph = pb[:, (s * HEADS + h) * BLOCK:(s * HEADS + h + 1) * BLOCK]
            vh = vs[:, h * DH:(h + 1) * DH]
            heads.append(jax.lax.dot_general(
                ph, vh, (((0,), (0,)), ((), ())),
                preferred_element_type=f32))
        row_blocks.append(jnp.concatenate(heads, axis=-1))
    att = jnp.concatenate(row_blocks, axis=0)
    o = jnp.dot(att.astype(bf16), wo_ref[0],
                preferred_element_type=f32) + bo_ref[0]
    o_ref[0] = o.astype(o_ref.dtype)


def _block_attn(x, wqkv, wo, bqkv, bo):
    m, s, _ = x.shape
    ntb = s // TB
    return pl.pallas_call(
        _block_attn_kernel,
        grid=(m, ntb),
        in_specs=[
            pl.BlockSpec((1, TB, DIM), lambda i, j: (i, j, 0)),
            pl.BlockSpec((1, DIM, 3 * DIM), lambda i, j: (i, 0, 0)),
            pl.BlockSpec((1, DIM, DIM), lambda i, j: (i, 0, 0)),
            pl.BlockSpec((1, 1, 3 * DIM), lambda i, j: (i, 0, 0)),
            pl.BlockSpec((1, 1, DIM), lambda i, j: (i, 0, 0)),
        ],
        out_specs=pl.BlockSpec((1, TB, DIM), lambda i, j: (i, j, 0)),
        out_shape=jax.ShapeDtypeStruct((m, s, DIM), jnp.bfloat16),
    )(x, wqkv, wo, bqkv, bo)


def _fusion_kernel(a_ref, c_ref, w_ref, b_ref, o_ref):
    f32 = jnp.float32
    acc = jnp.dot(a_ref[0], w_ref[0], preferred_element_type=f32)
    for i in range(1, NMODS):
        acc += jnp.dot(a_ref[i], w_ref[i], preferred_element_type=f32)
    for i in range(NMODS):
        acc += jnp.dot(c_ref[i], w_ref[NMODS + i], preferred_element_type=f32)
    o_ref[...] = acc + b_ref[...]


def _fusion(a, c, wf, bf):
    s = a.shape[1]
    ftb = min(FTB, s)
    nt = s // ftb
    return pl.pallas_call(
        _fusion_kernel,
        grid=(nt,),
        in_specs=[
            pl.BlockSpec((NMODS, ftb, DIM), lambda i: (0, i, 0)),
            pl.BlockSpec((NMODS, ftb, DIM), lambda i: (0, i, 0)),
            pl.BlockSpec((2 * NMODS, DIM, DIM), lambda i: (0, 0, 0)),
            pl.BlockSpec((1, DIM), lambda i: (0, 0)),
        ],
        out_specs=pl.BlockSpec((ftb, DIM), lambda i: (i, 0)),
        out_shape=jax.ShapeDtypeStruct((s, DIM), jnp.float32),
    )(a, c, wf, bf)


def _attn_operands(plist):
    scale = 1.0 / math.sqrt(DH)
    wqkv = jnp.stack([
        jnp.concatenate([p["Wq"] * scale, p["Wk"], p["Wv"]], axis=1)
        for p in plist]).astype(jnp.bfloat16)
    wo = jnp.stack([p["Wo"] for p in plist]).astype(jnp.bfloat16)
    bqkv = jnp.stack([
        jnp.concatenate([p["bq"] * scale, p["bk"], p["bv"]])
        for p in plist]).astype(jnp.float32).reshape(len(plist), 1, 3 * DIM)
    bo = jnp.stack([p["bo"] for p in plist]).astype(
        jnp.float32).reshape(len(plist), 1, DIM)
    return wqkv, wo, bqkv, bo


def _pipeline(x, wqkv_m, wo_m, bqkv_m, bo_m,
              wqkv_c, wo_c, bqkv_c, bo_c, wf, bfus):
    s_local = x.shape[1]
    attended = _block_attn(x, wqkv_m, wo_m, bqkv_m, bo_m)
    cross = _block_attn(attended.reshape(1, NMODS * s_local, DIM),
                        wqkv_c, wo_c, bqkv_c, bo_c)
    cross = cross.reshape(NMODS, s_local, DIM)
    return _fusion(attended, cross, wf, bfus)


def kernel(text, visual, audio, params):
    bf16 = jnp.bfloat16
    x = jnp.stack([text[0], visual[0], audio[0]]).astype(bf16)
    mod_ops = _attn_operands([params[m + "_attn"]
                              for m in ("text", "visual", "audio")])
    cross_ops = _attn_operands([params["cross_attn"]])
    fw = params["fusion_weights"].astype(jnp.float32)
    scales = jnp.concatenate([fw, fw]).reshape(2 * NMODS, 1, 1)
    wf = (params["fusion_W"].reshape(2 * NMODS, DIM, DIM) * scales).astype(bf16)
    bfus = params["fusion_b"].astype(jnp.float32).reshape(1, DIM)

    out = _pipeline(x, *mod_ops, *cross_ops, wf, bfus)
    return out.reshape(1, SEQ, DIM)


# speedup vs baseline: 3.5516x; 1.0455x over previous
import math

import jax
import jax.numpy as jnp
from jax.experimental import pallas as pl

DIM = 1024
HEADS = 16
BLOCK = 128
DH = DIM // HEADS
SEQ = 2048
NMODS = 3

TB = 512
FTB = 512


def _block_attn_kernel(x_ref, wqkv_ref, wo_ref, bqkv_ref, bo_ref, o_ref):
    f32 = jnp.float32
    bf16 = jnp.bfloat16
    x = x_ref[0]
    qkv = jnp.dot(x, wqkv_ref[0], preferred_element_type=f32) + bqkv_ref[0]
    qb = qkv[:, :DIM].astype(bf16)
    kb = qkv[:, DIM:2 * DIM].astype(bf16)
    vb = qkv[:, 2 * DIM:].astype(bf16)
    nsb = TB // BLOCK
    scores = []
    for s in range(nsb):
        qs = qb[s * BLOCK:(s + 1) * BLOCK]
        ks = kb[s * BLOCK:(s + 1) * BLOCK]
        for h in range(HEADS):
            qh = qs[:, h * DH:(h + 1) * DH]
            kh = ks[:, h * DH:(h + 1) * DH]
            scores.append(jax.lax.dot_general(
                kh, qh, (((1,), (1,)), ((), ())),
                preferred_element_type=f32))
    sc = jnp.concatenate(scores, axis=1)
    m = jnp.max(sc, axis=0, keepdims=True)
    e = jnp.exp(sc - m)
    p = e * (1.0 / jnp.sum(e, axis=0, keepdims=True))
    pb = p.astype(bf16)
    row_blocks = []
    for s in range(nsb):
        vs = vb[s * BLOCK:(s + 1) * BLOCK]
        heads = []
        for h in range(HEADS):
            ph = pb[:, (s * HEADS + h) * BLOCK:(s * HEADS + h + 1) * BLOCK]
            vh = vs[:, h * DH:(h + 1) * DH]
            heads.append(jax.lax.dot_general(
                ph, vh, (((0,), (0,)), ((), ())),
                preferred_element_type=f32))
        row_blocks.append(jnp.concatenate(heads, axis=-1))
    att = jnp.concatenate(row_blocks, axis=0)
    o = jnp.dot(att.astype(bf16), wo_ref[0],
                preferred_element_type=f32) + bo_ref[0]
    o_ref[0] = o.astype(o_ref.dtype)


def _block_attn(x, wqkv, wo, bqkv, bo):
    m, s, _ = x.shape
    ntb = s // TB
    return pl.pallas_call(
        _block_attn_kernel,
        grid=(m, ntb),
        in_specs=[
            pl.BlockSpec((1, TB, DIM), lambda i, j: (i, j, 0)),
            pl.BlockSpec((1, DIM, 3 * DIM), lambda i, j: (i, 0, 0)),
            pl.BlockSpec((1, DIM, DIM), lambda i, j: (i, 0, 0)),
            pl.BlockSpec((1, 1, 3 * DIM), lambda i, j: (i, 0, 0)),
            pl.BlockSpec((1, 1, DIM), lambda i, j: (i, 0, 0)),
        ],
        out_specs=pl.BlockSpec((1, TB, DIM), lambda i, j: (i, j, 0)),
        out_shape=jax.ShapeDtypeStruct((m, s, DIM), jnp.bfloat16),
    )(x, wqkv, wo, bqkv, bo)


def _fusion_kernel(a_ref, c_ref, w_ref, b_ref, o_ref):
    f32 = jnp.float32
    acc = jnp.dot(a_ref[0], w_ref[0], preferred_element_type=f32)
    for i in range(1, NMODS):
        acc += jnp.dot(a_ref[i], w_ref[i], preferred_element_type=f32)
    for i in range(NMODS):
        acc += jnp.dot(c_ref[i], w_ref[NMODS + i], preferred_element_type=f32)
    o_ref[...] = acc + b_ref[...]


def _fusion(a, c, wf, bf):
    s = a.shape[1]
    ftb = min(FTB, s)
    nt = s // ftb
    return pl.pallas_call(
        _fusion_kernel,
        grid=(nt,),
        in_specs=[
            pl.BlockSpec((NMODS, ftb, DIM), lambda i: (0, i, 0)),
            pl.BlockSpec((NMODS, ftb, DIM), lambda i: (0, i, 0)),
            pl.BlockSpec((2 * NMODS, DIM, DIM), lambda i: (0, 0, 0)),
            pl.BlockSpec((1, DIM), lambda i: (0, 0)),
        ],
        out_specs=pl.BlockSpec((ftb, DIM), lambda i: (i, 0)),
        out_shape=jax.ShapeDtypeStruct((s, DIM), jnp.float32),
    )(a, c, wf, bf)


def _attn_operands(plist):
    scale = 1.0 / math.sqrt(DH)
    wqkv = jnp.stack([
        jnp.concatenate([p["Wq"] * scale, p["Wk"], p["Wv"]], axis=1)
        for p in plist]).astype(jnp.bfloat16)
    wo = jnp.stack([p["Wo"] for p in plist]).astype(jnp.bfloat16)
    bqkv = jnp.stack([
        jnp.concatenate([p["bq"] * scale, p["bk"], p["bv"]])
        for p in plist]).astype(jnp.float32).reshape(len(plist), 1, 3 * DIM)
    bo = jnp.stack([p["bo"] for p in plist]).astype(
        jnp.float32).reshape(len(plist), 1, DIM)
    return wqkv, wo, bqkv, bo


def _pipeline(x, wqkv_m, wo_m, bqkv_m, bo_m,
              wqkv_c, wo_c, bqkv_c, bo_c, wf, bfus):
    s_local = x.shape[1]
    attended = _block_attn(x, wqkv_m, wo_m, bqkv_m, bo_m)
    cross = _block_attn(attended.reshape(1, NMODS * s_local, DIM),
                        wqkv_c, wo_c, bqkv_c, bo_c)
    cross = cross.reshape(NMODS, s_local, DIM)
    return _fusion(attended, cross, wf, bfus)


def kernel(text, visual, audio, params):
    bf16 = jnp.bfloat16
    x = jnp.stack([text[0], visual[0], audio[0]]).astype(bf16)
    mod_ops = _attn_operands([params[m + "_attn"]
                              for m in ("text", "visual", "audio")])
    cross_ops = _attn_operands([params["cross_attn"]])
    fw = params["fusion_weights"].astype(jnp.float32)
    scales = jnp.concatenate([fw, fw]).reshape(2 * NMODS, 1, 1)
    wf = (params["fusion_W"].reshape(2 * NMODS, DIM, DIM) * scales).astype(bf16)
    bfus = params["fusion_b"].astype(jnp.float32).reshape(1, DIM)

    out = _pipeline(x, *mod_ops, *cross_ops, wf, bfus)
    return out.reshape(1, SEQ, DIM)
